# baseline (speedup 1.0000x reference)
import jax
import jax.numpy as jnp
from jax import lax
from jax.experimental import pallas as pl
from jax.experimental.pallas import tpu as pltpu
from jax.experimental.pallas import tpu_sc as plsc

P, S, H, D, T = 512, 16, 8, 128, 2048
NKEY = P * S
NC, NS = 2, 16
NW = NC * NS
KEYS_PER_W = NKEY // NW
NSCAN = T // 16
CKEYS = 8
NBODY = KEYS_PER_W // (4 * CKEYS)


def _gather16(x, idx):
    dn = lax.GatherDimensionNumbers(
        offset_dims=(), collapsed_slice_dims=(0,), start_index_map=(0,))
    return lax.gather(x, idx.reshape(16, 1), dn, (1,),
                      mode=lax.GatherScatterMode.PROMISE_IN_BOUNDS)


def _body(kv_hbm, tp_hbm, ts_hbm, nk_hbm, nv_hbm, out_hbm,
          tp_v, ts_v, last_v, a0_v, a1_v, b0_v, b1_v,
          ra0, ra1, rb0, rb1, wa0, wa1, wb0, wb1, ua0, ua1, ub0, ub1):
    wid = lax.axis_index("s") * NC + lax.axis_index("c")
    key0 = wid * KEYS_PER_W

    def ckey(c):
        return key0 + c * CKEYS

    pltpu.async_copy(kv_hbm.at[pl.ds(ckey(0), CKEYS)], a0_v, ra0)
    pltpu.async_copy(kv_hbm.at[pl.ds(ckey(1), CKEYS)], a1_v, ra1)
    pltpu.async_copy(kv_hbm.at[pl.ds(ckey(2), CKEYS)], b0_v, rb0)
    pltpu.async_copy(kv_hbm.at[pl.ds(ckey(3), CKEYS)], b1_v, rb1)

    pltpu.sync_copy(tp_hbm, tp_v)
    pltpu.sync_copy(ts_hbm, ts_v)

    iota = lax.iota(jnp.int32, 16)

    def init_body(i, _):
        last_v[pl.ds(i * 16, 16)] = jnp.full((16,), -1, jnp.int32)
        return 0
    lax.fori_loop(0, NKEY // 16, init_body, 0)

    def scan_body(i, _):
        base = i * 16
        p = tp_v[pl.ds(base, 16)]
        s = ts_v[pl.ds(base, 16)]
        key = p * S + s
        loser = iota < 0
        for d in range(1, 16):
            shifted = _gather16(key, jnp.minimum(iota + d, 15))
            loser = loser | ((key == shifted) & (iota + d <= 15))
        plsc.store_scatter(last_v, [key], base + iota, mask=~loser)
        return 0
    lax.fori_loop(0, NSCAN, scan_body, 0)

    def drain_w(buf, sem):
        pltpu.make_async_copy(
            buf, out_hbm.at[pl.ds(key0, CKEYS)], sem).wait()

    def apply_updates(buf, usem, wv, lane0):
        n = jnp.int32(0)
        for sl in range(CKEYS):
            tok = wv[lane0 + sl]

            @pl.when(tok >= 0)
            def _upd():
                pltpu.async_copy(nk_hbm.at[tok], buf.at[sl, :, 0], usem)
                pltpu.async_copy(nv_hbm.at[tok], buf.at[sl, :, 1], usem)
            n = n + jnp.where(tok >= 0, jnp.int32(2), jnp.int32(0))

        def dr(j, _):
            pltpu.make_async_copy(nk_hbm.at[0], buf.at[0, :, 0], usem).wait()
            return 0
        lax.fori_loop(0, n, dr, 0)

    def proc_chunk(c, buf, rsem, wsem, usem, wv, lane0):
        pltpu.make_async_copy(
            kv_hbm.at[pl.ds(key0, CKEYS)], buf, rsem).wait()
        apply_updates(buf, usem, wv, lane0)
        pltpu.async_copy(buf, out_hbm.at[pl.ds(ckey(c), CKEYS)], wsem)

    def copy_body(i, _):
        c = i * 4

        @pl.when(i > 0)
        def _refire_b():
            drain_w(b0_v, wb0)
            pltpu.async_copy(kv_hbm.at[pl.ds(ckey(c + 2), CKEYS)], b0_v, rb0)
            drain_w(b1_v, wb1)
            pltpu.async_copy(kv_hbm.at[pl.ds(ckey(c + 3), CKEYS)], b1_v, rb1)

        wva = last_v[pl.ds(ckey(c), 16)]
        wvb = last_v[pl.ds(ckey(c + 2), 16)]
        proc_chunk(c, a0_v, ra0, wa0, ua0, wva, 0)
        proc_chunk(c + 1, a1_v, ra1, wa1, ua1, wva, 8)
        proc_chunk(c + 2, b0_v, rb0, wb0, ub0, wvb, 0)
        proc_chunk(c + 3, b1_v, rb1, wb1, ub1, wvb, 8)

        @pl.when(i < NBODY - 1)
        def _refire_a():
            drain_w(a0_v, wa0)
            pltpu.async_copy(kv_hbm.at[pl.ds(ckey(c + 4), CKEYS)], a0_v, ra0)
            drain_w(a1_v, wa1)
            pltpu.async_copy(kv_hbm.at[pl.ds(ckey(c + 5), CKEYS)], a1_v, ra1)
        return 0
    lax.fori_loop(0, NBODY, copy_body, 0)

    drain_w(a0_v, wa0)
    drain_w(a1_v, wa1)
    drain_w(b0_v, wb0)
    drain_w(b1_v, wb1)


@jax.jit
def _kv_scatter(kv4, t_pages, t_slots, nk3, nv3):
    mesh = plsc.VectorSubcoreMesh(core_axis_name="c", subcore_axis_name="s")
    return pl.kernel(
        _body,
        out_type=jax.ShapeDtypeStruct((NKEY, H, 2, D), jnp.float32),
        mesh=mesh,
        compiler_params=pltpu.CompilerParams(needs_layout_passes=False),
        scratch_types=[
            pltpu.VMEM((T,), jnp.int32),
            pltpu.VMEM((T,), jnp.int32),
            pltpu.VMEM((NKEY,), jnp.int32),
            pltpu.VMEM((CKEYS, H, 2, D), jnp.float32),
            pltpu.VMEM((CKEYS, H, 2, D), jnp.float32),
            pltpu.VMEM((CKEYS, H, 2, D), jnp.float32),
            pltpu.VMEM((CKEYS, H, 2, D), jnp.float32),
        ] + [pltpu.SemaphoreType.DMA] * 12,
    )(kv4, t_pages, t_slots, nk3, nv3)


def kernel(kv_pages, t_pages, t_slots, new_k, new_v, K):
    del K
    kv4 = kv_pages.reshape(NKEY, H, 2, D)
    out = _kv_scatter(kv4, t_pages.astype(jnp.int32),
                      t_slots.astype(jnp.int32),
                      new_k.astype(jnp.float32), new_v.astype(jnp.float32))
    return out.reshape(P, S, 2 * H, D)


# speedup vs baseline: 1.0185x; 1.0185x over previous
import jax
import jax.numpy as jnp
from jax import lax
from jax.experimental import pallas as pl
from jax.experimental.pallas import tpu as pltpu
from jax.experimental.pallas import tpu_sc as plsc

P, S, H, D, T = 512, 16, 8, 128, 2048
NK = S * 2 * H
ROWS = P * NK
NKEY = P * S
NC, NS = 2, 16
NW = NC * NS
KEYS_PER_W = NKEY // NW
ROWS_PER_W = ROWS // NW
NSCAN = T // 16
CROWS = 128
NBODY = ROWS_PER_W // (4 * CROWS)
SCANS_PER_IT = NSCAN // NBODY


def _gather16(x, idx):
    dn = lax.GatherDimensionNumbers(
        offset_dims=(), collapsed_slice_dims=(0,), start_index_map=(0,))
    return lax.gather(x, idx.reshape(16, 1), dn, (1,),
                      mode=lax.GatherScatterMode.PROMISE_IN_BOUNDS)


def _body(kv_hbm, tp_hbm, ts_hbm, nk_hbm, nv_hbm, out_hbm,
          tp_v, ts_v, last_v, tokl_v, keyl_v,
          idxsa_v, idxdka_v, idxdva_v, idxsb_v, idxdkb_v, idxdvb_v,
          sa0_v, sa1_v, sb0_v, sb1_v, spm_v,
          ra0, ra1, rb0, rb1, wa0, wa1, wb0, wb1):
    wid = lax.axis_index("s") * NC + lax.axis_index("c")
    sid = lax.axis_index("s")
    row0 = wid * ROWS_PER_W

    a0_v = spm_v.at[0]
    a1_v = spm_v.at[1]
    b0_v = spm_v.at[2]
    b1_v = spm_v.at[3]

    def crow(c):
        return row0 + c * CROWS

    pltpu.async_copy(kv_hbm.at[pl.ds(crow(0), CROWS)], a0_v, ra0)
    pltpu.async_copy(kv_hbm.at[pl.ds(crow(1), CROWS)], a1_v, ra1)
    pltpu.async_copy(kv_hbm.at[pl.ds(crow(2), CROWS)], b0_v, rb0)
    pltpu.async_copy(kv_hbm.at[pl.ds(crow(3), CROWS)], b1_v, rb1)

    pltpu.sync_copy(tp_hbm, tp_v)
    pltpu.sync_copy(ts_hbm, ts_v)

    iota = lax.iota(jnp.int32, 16)

    def init_body(i, _):
        last_v[pl.ds(i * 16, 16)] = jnp.full((16,), -1, jnp.int32)
        return 0
    lax.fori_loop(0, NKEY // 16, init_body, 0)

    def scan_step(i):
        base = i * 16
        p = tp_v[pl.ds(base, 16)]
        s = ts_v[pl.ds(base, 16)]
        key = p * S + s
        loser = iota < 0
        for d in range(1, 16):
            shifted = _gather16(key, jnp.minimum(iota + d, 15))
            loser = loser | ((key == shifted) & (iota + d <= 15))
        plsc.store_scatter(last_v, [key], base + iota, mask=~loser)

    def drain(buf, sem):
        pltpu.make_async_copy(kv_hbm.at[pl.ds(row0, CROWS)], buf, sem).wait()

    def drain_w(buf, sem):
        pltpu.make_async_copy(buf, out_hbm.at[pl.ds(row0, CROWS)], sem).wait()

    def scan_all(i, _):
        scan_step(i)
        return 0
    lax.fori_loop(0, NSCAN, scan_all, 0)

    def copy_body(i, _):
        c = i * 4

        @pl.when(i > 0)
        def _refire_b():
            drain_w(b0_v, wb0)
            pltpu.async_copy(kv_hbm.at[pl.ds(crow(c + 2), CROWS)], b0_v, rb0)
            drain_w(b1_v, wb1)
            pltpu.async_copy(kv_hbm.at[pl.ds(crow(c + 3), CROWS)], b1_v, rb1)

        drain(a0_v, ra0)
        pltpu.async_copy(a0_v, out_hbm.at[pl.ds(crow(c), CROWS)], wa0)
        drain(a1_v, ra1)
        pltpu.async_copy(a1_v, out_hbm.at[pl.ds(crow(c + 1), CROWS)], wa1)


        drain(b0_v, rb0)
        pltpu.async_copy(b0_v, out_hbm.at[pl.ds(crow(c + 2), CROWS)], wb0)
        drain(b1_v, rb1)
        pltpu.async_copy(b1_v, out_hbm.at[pl.ds(crow(c + 3), CROWS)], wb1)

        @pl.when(i < NBODY - 1)
        def _refire_a():
            drain_w(a0_v, wa0)
            pltpu.async_copy(kv_hbm.at[pl.ds(crow(c + 4), CROWS)], a0_v, ra0)
            drain_w(a1_v, wa1)
            pltpu.async_copy(kv_hbm.at[pl.ds(crow(c + 5), CROWS)], a1_v, ra1)
        return 0
    lax.fori_loop(0, NBODY, copy_body, 0)

    key0 = wid * KEYS_PER_W

    def sel_body(c, carry):
        off, best = carry
        kvec = key0 + c * 16 + iota
        wtok = plsc.load_gather(last_v, [kvec])
        m = wtok >= 0
        cnt = jnp.max(plsc.all_reduce_population_count(m))
        plsc.store_compressed(tokl_v.at[pl.ds(off, 16)], wtok, mask=m)
        plsc.store_compressed(keyl_v.at[pl.ds(off, 16)], kvec, mask=m)
        vbest = jnp.max(jnp.where(m, kvec * T + wtok, -1))
        return off + cnt, jnp.maximum(best, vbest)

    w_cnt, best = lax.fori_loop(0, KEYS_PER_W // 16, sel_body,
                                (jnp.int32(0), jnp.int32(-1)))

    @pl.when(w_cnt > 0)
    def _pad():
        ptok = jnp.full((16,), 1, jnp.int32) * (best & (T - 1))
        pkey = jnp.full((16,), 1, jnp.int32) * (best >> 11)
        tokl_v[pl.ds(w_cnt, 16)] = ptok
        keyl_v[pl.ds(w_cnt, 16)] = pkey
        tokl_v[pl.ds(w_cnt + 16, 16)] = ptok
        keyl_v[pl.ds(w_cnt + 16, 16)] = pkey

    drain_w(a0_v, wa0)
    drain_w(a1_v, wa1)
    drain_w(b0_v, wb0)
    drain_w(b1_v, wb1)

    def build_idx(ent, idxs, idxdk, idxdv):
        tok16 = tokl_v[pl.ds(ent, 16)]
        key16 = keyl_v[pl.ds(ent, 16)]
        for j in range(8):
            idxs[pl.ds(j * 16, 16)] = tok16 * H + j
            idxdk[pl.ds(j * 16, 16)] = key16 * (2 * H) + 2 * j
            idxdv[pl.ds(j * 16, 16)] = key16 * (2 * H) + 2 * j + 1

    npairs = (w_cnt + 31) // 32

    def dma_body(c, _):
        @pl.when(c > 0)
        def _drain_prev():
            drain_w(sa0_v, wa0)
            drain_w(sa1_v, wa1)
            drain_w(sb0_v, wb0)
            drain_w(sb1_v, wb1)

        build_idx(c * 32, idxsa_v, idxdka_v, idxdva_v)
        gk0 = pltpu.async_copy(nk_hbm.at[idxsa_v], sa0_v, ra0)
        gv0 = pltpu.async_copy(nv_hbm.at[idxsa_v], sa1_v, ra1)
        build_idx(c * 32 + 16, idxsb_v, idxdkb_v, idxdvb_v)
        gk1 = pltpu.async_copy(nk_hbm.at[idxsb_v], sb0_v, rb0)
        gv1 = pltpu.async_copy(nv_hbm.at[idxsb_v], sb1_v, rb1)
        gk0.wait()
        gv0.wait()
        pltpu.async_copy(sa0_v, out_hbm.at[idxdka_v], wa0)
        pltpu.async_copy(sa1_v, out_hbm.at[idxdva_v], wa1)
        gk1.wait()
        gv1.wait()
        pltpu.async_copy(sb0_v, out_hbm.at[idxdkb_v], wb0)
        pltpu.async_copy(sb1_v, out_hbm.at[idxdvb_v], wb1)
        return 0
    lax.fori_loop(0, npairs, dma_body, 0)

    @pl.when(w_cnt > 0)
    def _drain_last():
        drain_w(sa0_v, wa0)
        drain_w(sa1_v, wa1)
        drain_w(sb0_v, wb0)
        drain_w(sb1_v, wb1)


@jax.jit
def _kv_scatter(kv_flat, t_pages, t_slots, nk_flat, nv_flat):
    mesh = plsc.VectorSubcoreMesh(core_axis_name="c", subcore_axis_name="s")
    return pl.kernel(
        _body,
        out_type=jax.ShapeDtypeStruct((ROWS, D), jnp.float32),
        mesh=mesh,
        compiler_params=pltpu.CompilerParams(needs_layout_passes=False),
        scratch_types=[
            pltpu.VMEM((T,), jnp.int32),
            pltpu.VMEM((T,), jnp.int32),
            pltpu.VMEM((NKEY,), jnp.int32),
            pltpu.VMEM((KEYS_PER_W + 32,), jnp.int32),
            pltpu.VMEM((KEYS_PER_W + 32,), jnp.int32),
            pltpu.VMEM((128,), jnp.int32),
            pltpu.VMEM((128,), jnp.int32),
            pltpu.VMEM((128,), jnp.int32),
            pltpu.VMEM((128,), jnp.int32),
            pltpu.VMEM((128,), jnp.int32),
            pltpu.VMEM((128,), jnp.int32),
            pltpu.VMEM((128, D), jnp.float32),
            pltpu.VMEM((128, D), jnp.float32),
            pltpu.VMEM((128, D), jnp.float32),
            pltpu.VMEM((128, D), jnp.float32),
            pltpu.MemorySpace.VMEM_SHARED((4, CROWS, D), jnp.float32),
        ] + [pltpu.SemaphoreType.DMA] * 8,
    )(kv_flat, t_pages, t_slots, nk_flat, nv_flat)


def kernel(kv_pages, t_pages, t_slots, new_k, new_v, K):
    del K
    kv_flat = kv_pages.reshape(ROWS, D)
    nk_flat = new_k.astype(jnp.float32).reshape(T * H, D)
    nv_flat = new_v.astype(jnp.float32).reshape(T * H, D)
    out = _kv_scatter(kv_flat, t_pages.astype(jnp.int32),
                      t_slots.astype(jnp.int32), nk_flat, nv_flat)
    return out.reshape(P, S, 2 * H, D)


# speedup vs baseline: 1.0582x; 1.0390x over previous
import jax
import jax.numpy as jnp
from jax import lax
from jax.experimental import pallas as pl
from jax.experimental.pallas import tpu as pltpu
from jax.experimental.pallas import tpu_sc as plsc

P, S, H, D, T = 512, 16, 8, 128, 2048
NK = S * 2 * H
ROWS = P * NK
NKEY = P * S
NC, NS = 2, 16
NW = NC * NS
KEYS_PER_W = NKEY // NW
ROWS_PER_W = ROWS // NW
NSCAN = T // 16
CROWS = 128
NBODY = ROWS_PER_W // (4 * CROWS)
SCANS_PER_IT = NSCAN // NBODY


def _gather16(x, idx):
    dn = lax.GatherDimensionNumbers(
        offset_dims=(), collapsed_slice_dims=(0,), start_index_map=(0,))
    return lax.gather(x, idx.reshape(16, 1), dn, (1,),
                      mode=lax.GatherScatterMode.PROMISE_IN_BOUNDS)


def _body(kv_hbm, tp_hbm, ts_hbm, nk_hbm, nv_hbm, out_hbm,
          tp_v, ts_v, last_v, tokl_v, keyl_v,
          idxsa_v, idxdka_v, idxdva_v, idxsb_v, idxdkb_v, idxdvb_v,
          sa0_v, sa1_v, sb0_v, sb1_v, spm_v, wsc_v,
          ra0, ra1, rb0, rb1, wa0, wa1, wb0, wb1):
    wid = lax.axis_index("s") * NC + lax.axis_index("c")
    sid = lax.axis_index("s")
    row0 = wid * ROWS_PER_W

    a0_v = spm_v.at[0]
    a1_v = spm_v.at[1]
    b0_v = spm_v.at[2]
    b1_v = spm_v.at[3]

    def crow(c):
        return row0 + c * CROWS

    pltpu.async_copy(kv_hbm.at[pl.ds(crow(0), CROWS)], a0_v, ra0)
    pltpu.async_copy(kv_hbm.at[pl.ds(crow(1), CROWS)], a1_v, ra1)
    pltpu.async_copy(kv_hbm.at[pl.ds(crow(2), CROWS)], b0_v, rb0)
    pltpu.async_copy(kv_hbm.at[pl.ds(crow(3), CROWS)], b1_v, rb1)

    pltpu.sync_copy(tp_hbm, tp_v)
    pltpu.sync_copy(ts_hbm, ts_v)

    iota = lax.iota(jnp.int32, 16)

    def init_body(i, _):
        last_v[pl.ds(i * 16, 16)] = jnp.full((16,), -1, jnp.int32)
        return 0
    lax.fori_loop(0, NKEY // 16, init_body, 0)

    def scan_step(i):
        base = i * 16
        p = tp_v[pl.ds(base, 16)]
        s = ts_v[pl.ds(base, 16)]
        key = p * S + s
        loser = iota < 0
        for d in range(1, 16):
            shifted = _gather16(key, jnp.minimum(iota + d, 15))
            loser = loser | ((key == shifted) & (iota + d <= 15))
        plsc.store_scatter(last_v, [key], base + iota, mask=~loser)

    def drain(buf, sem):
        pltpu.make_async_copy(kv_hbm.at[pl.ds(row0, CROWS)], buf, sem).wait()

    def drain_w(buf, sem):
        pltpu.make_async_copy(buf, out_hbm.at[pl.ds(row0, CROWS)], sem).wait()

    key0 = wid * KEYS_PER_W

    def build_idx(ent, idxs, idxdk, idxdv):
        tok16 = tokl_v[pl.ds(ent, 16)]
        key16 = keyl_v[pl.ds(ent, 16)]
        for j in range(8):
            idxs[pl.ds(j * 16, 16)] = tok16 * H + j
            idxdk[pl.ds(j * 16, 16)] = key16 * (2 * H) + 2 * j
            idxdv[pl.ds(j * 16, 16)] = key16 * (2 * H) + 2 * j + 1

    def copy_body(i, _):
        c = i * 4

        @pl.when(i > 0)
        def _refire_b():
            drain_w(b0_v, wb0)
            pltpu.async_copy(kv_hbm.at[pl.ds(crow(c + 2), CROWS)], b0_v, rb0)
            drain_w(b1_v, wb1)
            pltpu.async_copy(kv_hbm.at[pl.ds(crow(c + 3), CROWS)], b1_v, rb1)

        drain(a0_v, ra0)
        pltpu.async_copy(a0_v, out_hbm.at[pl.ds(crow(c), CROWS)], wa0)
        drain(a1_v, ra1)
        pltpu.async_copy(a1_v, out_hbm.at[pl.ds(crow(c + 1), CROWS)], wa1)

        def scan_body(j, _):
            scan_step(i * SCANS_PER_IT + j)
            return 0
        lax.fori_loop(0, SCANS_PER_IT, scan_body, 0)

        drain(b0_v, rb0)
        pltpu.async_copy(b0_v, out_hbm.at[pl.ds(crow(c + 2), CROWS)], wb0)
        drain(b1_v, rb1)
        pltpu.async_copy(b1_v, out_hbm.at[pl.ds(crow(c + 3), CROWS)], wb1)

        @pl.when(i < NBODY - 1)
        def _refire_a():
            drain_w(a0_v, wa0)
            pltpu.async_copy(kv_hbm.at[pl.ds(crow(c + 4), CROWS)], a0_v, ra0)
            drain_w(a1_v, wa1)
            pltpu.async_copy(kv_hbm.at[pl.ds(crow(c + 5), CROWS)], a1_v, ra1)

        @pl.when(i == NBODY - 1)
        def _sel_tail():
            def sel_body(cc, carry):
                off, best = carry
                kvec = key0 + cc * 16 + iota
                wtok = plsc.load_gather(last_v, [kvec])
                m = wtok >= 0
                cnt = jnp.max(plsc.all_reduce_population_count(m))
                plsc.store_compressed(tokl_v.at[pl.ds(off, 16)], wtok, mask=m)
                plsc.store_compressed(keyl_v.at[pl.ds(off, 16)], kvec, mask=m)
                vbest = jnp.max(jnp.where(m, kvec * T + wtok, -1))
                return off + cnt, jnp.maximum(best, vbest)

            w_cnt, best = lax.fori_loop(0, KEYS_PER_W // 16, sel_body,
                                        (jnp.int32(0), jnp.int32(-1)))
            wsc_v[pl.ds(0, 16)] = jnp.full((16,), 1, jnp.int32) * w_cnt

            @pl.when(w_cnt > 0)
            def _pad():
                ptok = jnp.full((16,), 1, jnp.int32) * (best & (T - 1))
                pkey = jnp.full((16,), 1, jnp.int32) * (best >> 11)
                tokl_v[pl.ds(w_cnt, 16)] = ptok
                keyl_v[pl.ds(w_cnt, 16)] = pkey
                tokl_v[pl.ds(w_cnt + 16, 16)] = ptok
                keyl_v[pl.ds(w_cnt + 16, 16)] = pkey
                build_idx(0, idxsa_v, idxdka_v, idxdva_v)
                build_idx(16, idxsb_v, idxdkb_v, idxdvb_v)
                pltpu.async_copy(nk_hbm.at[idxsa_v], sa0_v, ra0)
                pltpu.async_copy(nv_hbm.at[idxsa_v], sa1_v, ra1)
                pltpu.async_copy(nk_hbm.at[idxsb_v], sb0_v, rb0)
                pltpu.async_copy(nv_hbm.at[idxsb_v], sb1_v, rb1)
        return 0
    lax.fori_loop(0, NBODY, copy_body, 0)

    w_cnt = wsc_v[pl.ds(0, 16)][0]

    drain_w(a0_v, wa0)
    drain_w(a1_v, wa1)
    drain_w(b0_v, wb0)
    drain_w(b1_v, wb1)

    npairs = (w_cnt + 31) // 32

    def gdrain():
        pltpu.make_async_copy(nk_hbm.at[pl.ds(0, 128)], sa0_v, ra0).wait()
        pltpu.make_async_copy(nv_hbm.at[pl.ds(0, 128)], sa1_v, ra1).wait()
        pltpu.make_async_copy(nk_hbm.at[pl.ds(0, 128)], sb0_v, rb0).wait()
        pltpu.make_async_copy(nv_hbm.at[pl.ds(0, 128)], sb1_v, rb1).wait()

    def sdrain():
        drain_w(sa0_v, wa0)
        drain_w(sa1_v, wa1)
        drain_w(sb0_v, wb0)
        drain_w(sb1_v, wb1)

    def dma_body(c, _):
        @pl.when(c > 0)
        def _next_gathers():
            sdrain()
            build_idx(c * 32, idxsa_v, idxdka_v, idxdva_v)
            build_idx(c * 32 + 16, idxsb_v, idxdkb_v, idxdvb_v)
            pltpu.async_copy(nk_hbm.at[idxsa_v], sa0_v, ra0)
            pltpu.async_copy(nv_hbm.at[idxsa_v], sa1_v, ra1)
            pltpu.async_copy(nk_hbm.at[idxsb_v], sb0_v, rb0)
            pltpu.async_copy(nv_hbm.at[idxsb_v], sb1_v, rb1)

        gdrain()
        pltpu.async_copy(sa0_v, out_hbm.at[idxdka_v], wa0)
        pltpu.async_copy(sa1_v, out_hbm.at[idxdva_v], wa1)
        pltpu.async_copy(sb0_v, out_hbm.at[idxdkb_v], wb0)
        pltpu.async_copy(sb1_v, out_hbm.at[idxdvb_v], wb1)
        return 0
    lax.fori_loop(0, npairs, dma_body, 0)

    @pl.when(w_cnt > 0)
    def _drain_last():
        sdrain()


@jax.jit
def _kv_scatter(kv_flat, t_pages, t_slots, nk_flat, nv_flat):
    mesh = plsc.VectorSubcoreMesh(core_axis_name="c", subcore_axis_name="s")
    return pl.kernel(
        _body,
        out_type=jax.ShapeDtypeStruct((ROWS, D), jnp.float32),
        mesh=mesh,
        compiler_params=pltpu.CompilerParams(needs_layout_passes=False),
        scratch_types=[
            pltpu.VMEM((T,), jnp.int32),
            pltpu.VMEM((T,), jnp.int32),
            pltpu.VMEM((NKEY,), jnp.int32),
            pltpu.VMEM((KEYS_PER_W + 32,), jnp.int32),
            pltpu.VMEM((KEYS_PER_W + 32,), jnp.int32),
            pltpu.VMEM((128,), jnp.int32),
            pltpu.VMEM((128,), jnp.int32),
            pltpu.VMEM((128,), jnp.int32),
            pltpu.VMEM((128,), jnp.int32),
            pltpu.VMEM((128,), jnp.int32),
            pltpu.VMEM((128,), jnp.int32),
            pltpu.VMEM((128, D), jnp.float32),
            pltpu.VMEM((128, D), jnp.float32),
            pltpu.VMEM((128, D), jnp.float32),
            pltpu.VMEM((128, D), jnp.float32),
            pltpu.MemorySpace.VMEM_SHARED((4, CROWS, D), jnp.float32),
            pltpu.VMEM((16,), jnp.int32),
        ] + [pltpu.SemaphoreType.DMA] * 8,
    )(kv_flat, t_pages, t_slots, nk_flat, nv_flat)


def kernel(kv_pages, t_pages, t_slots, new_k, new_v, K):
    del K
    kv_flat = kv_pages.reshape(ROWS, D)
    nk_flat = new_k.astype(jnp.float32).reshape(T * H, D)
    nv_flat = new_v.astype(jnp.float32).reshape(T * H, D)
    out = _kv_scatter(kv_flat, t_pages.astype(jnp.int32),
                      t_slots.astype(jnp.int32), nk_flat, nv_flat)
    return out.reshape(P, S, 2 * H, D)


# speedup vs baseline: 1.0671x; 1.0084x over previous
import jax
import jax.numpy as jnp
from jax import lax
from jax.experimental import pallas as pl
from jax.experimental.pallas import tpu as pltpu
from jax.experimental.pallas import tpu_sc as plsc

P, S, H, D, T = 512, 16, 8, 128, 2048
NK = S * 2 * H
ROWS = P * NK
NKEY = P * S
NC, NS = 2, 16
NW = NC * NS
KEYS_PER_W = NKEY // NW
ROWS_PER_W = ROWS // NW
NSCAN = T // 16
CROWS = 128
NBODY = ROWS_PER_W // (4 * CROWS)
SCANS_PER_IT = NSCAN // NBODY


def _gather16(x, idx):
    dn = lax.GatherDimensionNumbers(
        offset_dims=(), collapsed_slice_dims=(0,), start_index_map=(0,))
    return lax.gather(x, idx.reshape(16, 1), dn, (1,),
                      mode=lax.GatherScatterMode.PROMISE_IN_BOUNDS)


def _body(kv_hbm, tp_hbm, ts_hbm, nk_hbm, nv_hbm, out_hbm,
          tp_v, ts_v, last_v, tokl_v, keyl_v,
          idxsa_v, idxdka_v, idxdva_v, idxsb_v, idxdkb_v, idxdvb_v,
          sa0_v, sa1_v, sb0_v, sb1_v, spm_v,
          ra0, ra1, rb0, rb1, wa0, wa1, wb0, wb1):
    wid = lax.axis_index("s") * NC + lax.axis_index("c")
    sid = lax.axis_index("s")
    row0 = wid * ROWS_PER_W

    a0_v = spm_v.at[0]
    a1_v = spm_v.at[1]
    b0_v = spm_v.at[2]
    b1_v = spm_v.at[3]

    def crow(c):
        return row0 + c * CROWS

    pltpu.async_copy(kv_hbm.at[pl.ds(crow(0), CROWS)], a0_v, ra0)
    pltpu.async_copy(kv_hbm.at[pl.ds(crow(1), CROWS)], a1_v, ra1)
    pltpu.async_copy(kv_hbm.at[pl.ds(crow(2), CROWS)], b0_v, rb0)
    pltpu.async_copy(kv_hbm.at[pl.ds(crow(3), CROWS)], b1_v, rb1)

    pltpu.sync_copy(tp_hbm, tp_v)
    pltpu.sync_copy(ts_hbm, ts_v)

    iota = lax.iota(jnp.int32, 16)

    def init_body(i, _):
        last_v[pl.ds(i * 16, 16)] = jnp.full((16,), -1, jnp.int32)
        return 0
    lax.fori_loop(0, NKEY // 16, init_body, 0)

    def scan_step(i):
        base = i * 16
        p = tp_v[pl.ds(base, 16)]
        s = ts_v[pl.ds(base, 16)]
        key = p * S + s
        loser = iota < 0
        for d in range(1, 16):
            shifted = _gather16(key, jnp.minimum(iota + d, 15))
            loser = loser | ((key == shifted) & (iota + d <= 15))
        plsc.store_scatter(last_v, [key], base + iota, mask=~loser)

    def drain(buf, sem):
        pltpu.make_async_copy(kv_hbm.at[pl.ds(row0, CROWS)], buf, sem).wait()

    def drain_w(buf, sem):
        pltpu.make_async_copy(buf, out_hbm.at[pl.ds(row0, CROWS)], sem).wait()

    def copy_body(i, _):
        c = i * 4

        @pl.when(i > 0)
        def _refire_b():
            drain_w(b0_v, wb0)
            pltpu.async_copy(kv_hbm.at[pl.ds(crow(c + 2), CROWS)], b0_v, rb0)
            drain_w(b1_v, wb1)
            pltpu.async_copy(kv_hbm.at[pl.ds(crow(c + 3), CROWS)], b1_v, rb1)

        drain(a0_v, ra0)
        pltpu.async_copy(a0_v, out_hbm.at[pl.ds(crow(c), CROWS)], wa0)
        drain(a1_v, ra1)
        pltpu.async_copy(a1_v, out_hbm.at[pl.ds(crow(c + 1), CROWS)], wa1)

        def scan_body(j, _):
            scan_step(i * SCANS_PER_IT + j)
            return 0
        lax.fori_loop(0, SCANS_PER_IT, scan_body, 0)

        drain(b0_v, rb0)
        pltpu.async_copy(b0_v, out_hbm.at[pl.ds(crow(c + 2), CROWS)], wb0)
        drain(b1_v, rb1)
        pltpu.async_copy(b1_v, out_hbm.at[pl.ds(crow(c + 3), CROWS)], wb1)

        @pl.when(i < NBODY - 1)
        def _refire_a():
            drain_w(a0_v, wa0)
            pltpu.async_copy(kv_hbm.at[pl.ds(crow(c + 4), CROWS)], a0_v, ra0)
            drain_w(a1_v, wa1)
            pltpu.async_copy(kv_hbm.at[pl.ds(crow(c + 5), CROWS)], a1_v, ra1)
        return 0
    lax.fori_loop(0, NBODY, copy_body, 0)

    key0 = wid * KEYS_PER_W

    def sel_body(c, carry):
        off, best = carry
        kvec = key0 + c * 16 + iota
        wtok = plsc.load_gather(last_v, [kvec])
        m = wtok >= 0
        cnt = jnp.max(plsc.all_reduce_population_count(m))
        plsc.store_compressed(tokl_v.at[pl.ds(off, 16)], wtok, mask=m)
        plsc.store_compressed(keyl_v.at[pl.ds(off, 16)], kvec, mask=m)
        vbest = jnp.max(jnp.where(m, kvec * T + wtok, -1))
        return off + cnt, jnp.maximum(best, vbest)

    w_cnt, best = lax.fori_loop(0, KEYS_PER_W // 16, sel_body,
                                (jnp.int32(0), jnp.int32(-1)))

    @pl.when(w_cnt > 0)
    def _pad():
        ptok = jnp.full((16,), 1, jnp.int32) * (best & (T - 1))
        pkey = jnp.full((16,), 1, jnp.int32) * (best >> 11)
        tokl_v[pl.ds(w_cnt, 16)] = ptok
        keyl_v[pl.ds(w_cnt, 16)] = pkey
        tokl_v[pl.ds(w_cnt + 16, 16)] = ptok
        keyl_v[pl.ds(w_cnt + 16, 16)] = pkey

    drain_w(a0_v, wa0)
    drain_w(a1_v, wa1)
    drain_w(b0_v, wb0)
    drain_w(b1_v, wb1)

    def build_idx(ent, idxs, idxdk, idxdv):
        tok16 = tokl_v[pl.ds(ent, 16)]
        key16 = keyl_v[pl.ds(ent, 16)]
        for j in range(8):
            idxs[pl.ds(j * 16, 16)] = tok16 * H + j
            idxdk[pl.ds(j * 16, 16)] = key16 * (2 * H) + 2 * j
            idxdv[pl.ds(j * 16, 16)] = key16 * (2 * H) + 2 * j + 1

    npairs = (w_cnt + 31) // 32

    def dma_body(c, _):
        @pl.when(c > 0)
        def _drain_prev():
            drain_w(sa0_v, wa0)
            drain_w(sa1_v, wa1)
            drain_w(sb0_v, wb0)
            drain_w(sb1_v, wb1)

        build_idx(c * 32, idxsa_v, idxdka_v, idxdva_v)
        gk0 = pltpu.async_copy(nk_hbm.at[idxsa_v], sa0_v, ra0)
        gv0 = pltpu.async_copy(nv_hbm.at[idxsa_v], sa1_v, ra1)
        build_idx(c * 32 + 16, idxsb_v, idxdkb_v, idxdvb_v)
        gk1 = pltpu.async_copy(nk_hbm.at[idxsb_v], sb0_v, rb0)
        gv1 = pltpu.async_copy(nv_hbm.at[idxsb_v], sb1_v, rb1)
        gk0.wait()
        gv0.wait()
        pltpu.async_copy(sa0_v, out_hbm.at[idxdka_v], wa0)
        pltpu.async_copy(sa1_v, out_hbm.at[idxdva_v], wa1)
        gk1.wait()
        gv1.wait()
        pltpu.async_copy(sb0_v, out_hbm.at[idxdkb_v], wb0)
        pltpu.async_copy(sb1_v, out_hbm.at[idxdvb_v], wb1)
        return 0
    lax.fori_loop(0, npairs, dma_body, 0)

    @pl.when(w_cnt > 0)
    def _drain_last():
        drain_w(sa0_v, wa0)
        drain_w(sa1_v, wa1)
        drain_w(sb0_v, wb0)
        drain_w(sb1_v, wb1)


@jax.jit
def _kv_scatter(kv_flat, t_pages, t_slots, nk_flat, nv_flat):
    mesh = plsc.VectorSubcoreMesh(core_axis_name="c", subcore_axis_name="s")
    return pl.kernel(
        _body,
        out_type=jax.ShapeDtypeStruct((ROWS, D), jnp.float32),
        mesh=mesh,
        compiler_params=pltpu.CompilerParams(needs_layout_passes=False),
        scratch_types=[
            pltpu.VMEM((T,), jnp.int32),
            pltpu.VMEM((T,), jnp.int32),
            pltpu.VMEM((NKEY,), jnp.int32),
            pltpu.VMEM((KEYS_PER_W + 32,), jnp.int32),
            pltpu.VMEM((KEYS_PER_W + 32,), jnp.int32),
            pltpu.VMEM((128,), jnp.int32),
            pltpu.VMEM((128,), jnp.int32),
            pltpu.VMEM((128,), jnp.int32),
            pltpu.VMEM((128,), jnp.int32),
            pltpu.VMEM((128,), jnp.int32),
            pltpu.VMEM((128,), jnp.int32),
            pltpu.VMEM((128, D), jnp.float32),
            pltpu.VMEM((128, D), jnp.float32),
            pltpu.VMEM((128, D), jnp.float32),
            pltpu.VMEM((128, D), jnp.float32),
            pltpu.MemorySpace.VMEM_SHARED((4, CROWS, D), jnp.float32),
        ] + [pltpu.SemaphoreType.DMA] * 8,
    )(kv_flat, t_pages, t_slots, nk_flat, nv_flat)


def kernel(kv_pages, t_pages, t_slots, new_k, new_v, K):
    del K
    kv_flat = kv_pages.reshape(ROWS, D)
    nk_flat = new_k.astype(jnp.float32).reshape(T * H, D)
    nv_flat = new_v.astype(jnp.float32).reshape(T * H, D)
    out = _kv_scatter(kv_flat, t_pages.astype(jnp.int32),
                      t_slots.astype(jnp.int32), nk_flat, nv_flat)
    return out.reshape(P, S, 2 * H, D)
